# baseline (speedup 1.0000x reference)
import functools

import jax
import jax.numpy as jnp
from jax import lax
from jax.experimental import pallas as pl
from jax.experimental.pallas import tpu as pltpu
from jax.experimental.pallas import tpu_sc as plsc

R, C = 16384, 512
OC = C // 2
NW = 32
ROWS_PER_W = R // NW
N_CHUNK = 16
CH_ROWS = ROWS_PER_W // N_CHUNK
LANES = 16
VECS_PER_ROW = OC // LANES
N_IN = 4
N_OUT = 3

_mesh = plsc.VectorSubcoreMesh(core_axis_name="c", subcore_axis_name="s")


@functools.partial(
    pl.kernel,
    mesh=_mesh,
    out_type=jax.ShapeDtypeStruct((R, OC), jnp.float32),
    scratch_types=[
        *[pltpu.VMEM((CH_ROWS, C), jnp.float32) for _ in range(N_IN)],
        *[pltpu.VMEM((2 * CH_ROWS, OC), jnp.float32) for _ in range(N_OUT)],
        pltpu.SemaphoreType.DMA,
        pltpu.SemaphoreType.DMA,
    ],
    compiler_params=pltpu.CompilerParams(needs_layout_passes=False),
)
def _deinterleave(in_hbm, out_hbm, *refs):
    in_bufs = refs[0:N_IN]
    out_bufs = refs[N_IN:N_IN + N_OUT]
    in_sem, out_sem = refs[N_IN + N_OUT], refs[N_IN + N_OUT + 1]
    wid = lax.axis_index("s") * 2 + lax.axis_index("c")
    row_base = wid * ROWS_PER_W
    iota2 = lax.iota(jnp.int32, LANES) * 2

    def in_copy(c):
        return pltpu.async_copy(
            in_hbm.at[pl.ds(row_base + c * CH_ROWS, CH_ROWS), :],
            in_bufs[c % N_IN], in_sem)

    def out_copy(g):
        return pltpu.async_copy(
            out_bufs[g % N_OUT],
            out_hbm.at[pl.ds(row_base + g * 2 * CH_ROWS, 2 * CH_ROWS), :],
            out_sem)

    in_h = [in_copy(c) for c in range(N_IN)]
    out_h = [None] * N_OUT
    for c in range(N_CHUNK):
        g, half = c // 2, c % 2
        in_h[c % N_IN].wait()
        if half == 0 and out_h[g % N_OUT] is not None:
            out_h[g % N_OUT].wait()
        iv = in_bufs[c % N_IN]
        ov = out_bufs[g % N_OUT]

        @plsc.parallel_loop(0, CH_ROWS * VECS_PER_ROW, 1, unroll=4)
        def _(i):
            r = i >> 4
            j = i & (VECS_PER_ROW - 1)
            col = iota2 + j * (2 * LANES)
            row = jnp.full((LANES,), r, jnp.int32)
            ov[r + half * CH_ROWS, pl.ds(j * LANES, LANES)] = (
                plsc.load_gather(iv, [row, col]))

        if half == 1:
            out_h[g % N_OUT] = out_copy(g)
        if c + N_IN < N_CHUNK:
            in_h[c % N_IN] = in_copy(c + N_IN)
    for h in out_h:
        if h is not None:
            h.wait()


def kernel(inputs):
    return _deinterleave(inputs)


# speedup vs baseline: 1.0075x; 1.0075x over previous
import functools

import jax
import jax.numpy as jnp
from jax import lax
from jax.experimental import pallas as pl
from jax.experimental.pallas import tpu as pltpu
from jax.experimental.pallas import tpu_sc as plsc

R, C = 16384, 512
OC = C // 2
NW = 32
ROWS_PER_W = R // NW
N_CHUNK = 16
CH_ROWS = ROWS_PER_W // N_CHUNK
LANES = 16
VECS_PER_ROW = OC // LANES
N_IN = 4
N_OUT = 3

_mesh = plsc.VectorSubcoreMesh(core_axis_name="c", subcore_axis_name="s")


@functools.partial(
    pl.kernel,
    mesh=_mesh,
    out_type=jax.ShapeDtypeStruct((R, OC), jnp.float32),
    scratch_types=[
        *[pltpu.VMEM((CH_ROWS, C), jnp.float32) for _ in range(N_IN)],
        *[pltpu.VMEM((2 * CH_ROWS, OC), jnp.float32) for _ in range(N_OUT)],
        pltpu.SemaphoreType.DMA,
        pltpu.SemaphoreType.DMA,
    ],
    compiler_params=pltpu.CompilerParams(needs_layout_passes=False),
)
def _deinterleave(in_hbm, out_hbm, *refs):
    in_bufs = refs[0:N_IN]
    out_bufs = refs[N_IN:N_IN + N_OUT]
    in_sem, out_sem = refs[N_IN + N_OUT], refs[N_IN + N_OUT + 1]
    wid = lax.axis_index("s") * 2 + lax.axis_index("c")
    row_base = wid * ROWS_PER_W
    iota2 = lax.iota(jnp.int32, LANES) * 2

    def in_copy(c):
        return pltpu.async_copy(
            in_hbm.at[pl.ds(row_base + c * CH_ROWS, CH_ROWS), :],
            in_bufs[c % N_IN], in_sem)

    def out_copy(g):
        return pltpu.async_copy(
            out_bufs[g % N_OUT],
            out_hbm.at[pl.ds(row_base + g * 2 * CH_ROWS, 2 * CH_ROWS), :],
            out_sem)

    in_h = [in_copy(c) for c in range(N_IN)]
    out_h = [None] * N_OUT
    for c in range(N_CHUNK):
        g, half = c // 2, c % 2
        in_h[c % N_IN].wait()
        if half == 0 and out_h[g % N_OUT] is not None:
            out_h[g % N_OUT].wait()
        iv = in_bufs[c % N_IN]
        ov = out_bufs[g % N_OUT]

        @plsc.parallel_loop(0, CH_ROWS * VECS_PER_ROW, 1, unroll=8)
        def _(i):
            r = i >> 4
            j = i & (VECS_PER_ROW - 1)
            col = iota2 + j * (2 * LANES)
            row = jnp.full((LANES,), r, jnp.int32)
            ov[r + half * CH_ROWS, pl.ds(j * LANES, LANES)] = (
                plsc.load_gather(iv, [row, col]))

        if half == 1:
            out_h[g % N_OUT] = out_copy(g)
        if c + N_IN < N_CHUNK:
            in_h[c % N_IN] = in_copy(c + N_IN)
    for h in out_h:
        if h is not None:
            h.wait()


def kernel(inputs):
    return _deinterleave(inputs)
